# baseline (speedup 1.0000x reference)
import functools

import jax
import jax.numpy as jnp
from jax import lax
from jax.experimental import pallas as pl
from jax.experimental.pallas import tpu as pltpu
from jax.experimental.pallas import tpu_sc as plsc

N_CARDS = 52
DIM = 64
BATCH = 16384
N_IDX = 200
TOT = BATCH * N_IDX
NW = 32
PER_W = TOT // NW
CHUNK = 800
ITERS = PER_W // CHUNK

_mesh = plsc.VectorSubcoreMesh(core_axis_name="c", subcore_axis_name="s")


@functools.partial(
    pl.kernel,
    out_type=jax.ShapeDtypeStruct((TOT, DIM), jnp.float32),
    mesh=_mesh,
    scratch_types=[
        pltpu.VMEM((CHUNK,), jnp.int32),
        pltpu.VMEM((CHUNK,), jnp.int32),
        pltpu.VMEM((CHUNK, DIM), jnp.float32),
        pltpu.VMEM((CHUNK, DIM), jnp.float32),
        pltpu.SemaphoreType.DMA,
        pltpu.SemaphoreType.DMA,
        pltpu.SemaphoreType.DMA,
        pltpu.SemaphoreType.DMA,
        pltpu.SemaphoreType.DMA,
        pltpu.SemaphoreType.DMA,
    ],
    compiler_params=pltpu.CompilerParams(use_tc_tiling_on_sc=False),
)
def _emb_lookup(idx_hbm, table_hbm, out_hbm, idx0, idx1, rows0, rows1,
                si0, si1, sg0, sg1, ss0, ss1):
    wid = lax.axis_index("s") * 2 + lax.axis_index("c")
    base = wid * PER_W
    idx_v = (idx0, idx1)
    rows_v = (rows0, rows1)
    sem_i = (si0, si1)
    sem_g = (sg0, sg1)
    sem_s = (ss0, ss1)

    def idx_start(c, b):
        pltpu.async_copy(idx_hbm.at[pl.ds(base + c * CHUNK, CHUNK)],
                         idx_v[b], sem_i[b])

    def idx_wait(c, b):
        pltpu.make_async_copy(idx_hbm.at[pl.ds(base + c * CHUNK, CHUNK)],
                              idx_v[b], sem_i[b]).wait()

    def gather(b):
        pltpu.async_copy(table_hbm.at[idx_v[b]], rows_v[b], sem_g[b]).wait()

    def scatter_start(c, b):
        pltpu.async_copy(rows_v[b], out_hbm.at[pl.ds(base + c * CHUNK, CHUNK)],
                         sem_s[b])

    def scatter_wait(c, b):
        pltpu.make_async_copy(rows_v[b],
                              out_hbm.at[pl.ds(base + c * CHUNK, CHUNK)],
                              sem_s[b]).wait()

    idx_start(0, 0)
    idx_start(1, 1)
    for b in (0, 1):
        idx_wait(b, b)
        gather(b)
        scatter_start(b, b)
        idx_start(b + 2, b)

    def body(i, carry):
        for b in (0, 1):
            c = 2 * i + b
            idx_wait(c, b)
            scatter_wait(c - 2, b)
            gather(b)
            scatter_start(c, b)

            @pl.when(i < ITERS // 2 - 1)
            def _():
                idx_start(c + 2, b)

        return carry

    lax.fori_loop(1, ITERS // 2, body, 0)

    scatter_wait(ITERS - 2, 0)
    scatter_wait(ITERS - 1, 1)


def kernel(card_idxs, card_emb_weight):
    flat_idx = card_idxs.reshape(TOT)
    out = _emb_lookup(flat_idx, card_emb_weight)
    return out.reshape(BATCH, N_IDX, DIM)


# speedup vs baseline: 2.4758x; 2.4758x over previous
import functools

import jax
import jax.numpy as jnp
from jax import lax
from jax.experimental import pallas as pl
from jax.experimental.pallas import tpu as pltpu
from jax.experimental.pallas import tpu_sc as plsc

N_CARDS = 52
DIM = 64
BATCH = 16384
N_IDX = 200
TOT = BATCH * N_IDX
NW = 32
PER_W = TOT // NW
CHUNK = 800
ITERS = PER_W // CHUNK

_mesh = plsc.VectorSubcoreMesh(core_axis_name="c", subcore_axis_name="s")


@functools.partial(
    pl.kernel,
    out_type=jax.ShapeDtypeStruct((TOT, DIM), jnp.float32),
    mesh=_mesh,
    scratch_types=[
        pltpu.VMEM((CHUNK,), jnp.int32),
        pltpu.VMEM((CHUNK,), jnp.int32),
        pltpu.VMEM((CHUNK, DIM), jnp.float32),
        pltpu.VMEM((CHUNK, DIM), jnp.float32),
        pltpu.VMEM_SHARED((N_CARDS, DIM), jnp.float32),
        pltpu.SemaphoreType.DMA,
        pltpu.SemaphoreType.DMA,
        pltpu.SemaphoreType.DMA,
        pltpu.SemaphoreType.DMA,
        pltpu.SemaphoreType.DMA,
        pltpu.SemaphoreType.DMA,
    ],
    compiler_params=pltpu.CompilerParams(use_tc_tiling_on_sc=False),
)
def _emb_lookup(idx_hbm, table_hbm, out_hbm, idx0, idx1, rows0, rows1,
                table_sh, si0, si1, sg0, sg1, ss0, ss1):
    wid = lax.axis_index("s") * 2 + lax.axis_index("c")
    base = wid * PER_W

    @pl.when(lax.axis_index("s") == 0)
    def _():
        pltpu.sync_copy(table_hbm, table_sh)

    plsc.subcore_barrier()
    idx_v = (idx0, idx1)
    rows_v = (rows0, rows1)
    sem_i = (si0, si1)
    sem_g = (sg0, sg1)
    sem_s = (ss0, ss1)

    def idx_start(c, b):
        pltpu.async_copy(idx_hbm.at[pl.ds(base + c * CHUNK, CHUNK)],
                         idx_v[b], sem_i[b])

    def idx_wait(c, b):
        pltpu.make_async_copy(idx_hbm.at[pl.ds(base + c * CHUNK, CHUNK)],
                              idx_v[b], sem_i[b]).wait()

    def gather(b):
        pltpu.async_copy(table_sh.at[idx_v[b]], rows_v[b], sem_g[b]).wait()

    def scatter_start(c, b):
        pltpu.async_copy(rows_v[b], out_hbm.at[pl.ds(base + c * CHUNK, CHUNK)],
                         sem_s[b])

    def scatter_wait(c, b):
        pltpu.make_async_copy(rows_v[b],
                              out_hbm.at[pl.ds(base + c * CHUNK, CHUNK)],
                              sem_s[b]).wait()

    idx_start(0, 0)
    idx_start(1, 1)
    for b in (0, 1):
        idx_wait(b, b)
        gather(b)
        scatter_start(b, b)
        idx_start(b + 2, b)

    def body(i, carry):
        for b in (0, 1):
            c = 2 * i + b
            idx_wait(c, b)
            scatter_wait(c - 2, b)
            gather(b)
            scatter_start(c, b)

            @pl.when(i < ITERS // 2 - 1)
            def _():
                idx_start(c + 2, b)

        return carry

    lax.fori_loop(1, ITERS // 2, body, 0)

    scatter_wait(ITERS - 2, 0)
    scatter_wait(ITERS - 1, 1)


def kernel(card_idxs, card_emb_weight):
    flat_idx = card_idxs.reshape(TOT)
    out = _emb_lookup(flat_idx, card_emb_weight)
    return out.reshape(BATCH, N_IDX, DIM)
